# baseline (speedup 1.0000x reference)
import jax
import jax.numpy as jnp
from jax import lax
from jax.experimental import pallas as pl
from jax.experimental.pallas import tpu as pltpu
from jax.experimental.pallas import tpu_sc as plsc

_N, _L, _V = 8, 512, 32000
_B = _N * _L
_NC = 2
_NS = 16
_NW = _NC * _NS
_CHUNK = _B // _NW
_LANES = 16
_TILE_R, _TILE_C = 8, 128
_ROUND = 32
_NROUND = _CHUNK // _ROUND


def _partials_body(lp_hbm, tgt_hbm, msk_hbm, out_hbm,
                   tgt_v, msk_v, val_v, gran_a, gran_b, part_v,
                   sem_a, sem_b):
    wid = lax.axis_index("s") * _NC + lax.axis_index("c")
    n = wid & (_N - 1)
    lwin = lax.shift_right_logical(wid, 3)
    col_base = pl.multiple_of(lwin * _CHUNK, _CHUNK)
    base = n * _L + lwin * _CHUNK

    pltpu.sync_copy(tgt_hbm.at[:, pl.ds(col_base, _CHUNK)], tgt_v)
    pltpu.sync_copy(msk_hbm.at[:, pl.ds(col_base, _CHUNK)], msk_v)

    lane = lax.iota(jnp.int32, _LANES)
    bufs = (gran_a, gran_b)
    sems = (sem_a, sem_b)

    def fire(rnd):
        buf, sem = bufs[rnd % 2], sems[rnd % 2]
        copies = []
        for j in range(_ROUND // _LANES):
            gbase = rnd * _ROUND + j * _LANES
            tv = tgt_v[n, pl.ds(gbase, _LANES)]
            ctv = lax.shift_right_logical(tv, 7)
            for i in range(_LANES):
                g = gbase + i
                row0 = pl.multiple_of(base + g - (g % _TILE_R), _TILE_R)
                col0 = pl.multiple_of(ctv[i] * _TILE_C, _TILE_C)
                copies.append(pltpu.async_copy(
                    lp_hbm.at[pl.ds(row0, _TILE_R), pl.ds(col0, _TILE_C)],
                    buf.at[j * _LANES + i], sem))
        return copies

    def consume(rnd):
        buf = bufs[rnd % 2]
        for j in range(_ROUND // _LANES):
            gbase = rnd * _ROUND + j * _LANES
            tv = tgt_v[n, pl.ds(gbase, _LANES)]
            cols = tv & (_TILE_C - 1)
            vacc = jnp.zeros((_LANES,), jnp.float32)
            for i in range(_LANES):
                p = j * _LANES + i
                start = cols[i] - i
                v16 = buf[p, i & (_TILE_R - 1), pl.ds(start, _LANES)]
                vacc = jnp.where(lane == i, v16, vacc)
            val_v[pl.ds(gbase, _LANES)] = vacc

    pending = fire(0)
    for rnd in range(_NROUND):
        nxt = fire(rnd + 1) if rnd + 1 < _NROUND else ()
        for c in pending:
            c.wait()
        consume(rnd)
        pending = nxt

    accw = jnp.zeros((_LANES,), jnp.float32)
    accm = jnp.zeros((_LANES,), jnp.float32)
    for j in range(_CHUNK // _LANES):
        sl = pl.ds(j * _LANES, _LANES)
        m = msk_v[n, sl]
        accw = accw + val_v[sl] * m
        accm = accm + m
    part_v[0, :] = accw
    part_v[1, :] = accm
    pltpu.sync_copy(part_v, out_hbm.at[wid])


def _finalize_body(parts_ref, out_ref):
    ws = jnp.sum(parts_ref[:, 0, :])
    ms = jnp.sum(parts_ref[:, 1, :])
    out_ref[...] = jnp.full((1, 1), -ws / (ms + 1e-6), jnp.float32)


def kernel(logprobs, target, mask):
    lp = logprobs.reshape(_B, _V)
    tgt = target.astype(jnp.int32)
    msk = mask.astype(jnp.float32)

    mesh = plsc.VectorSubcoreMesh(core_axis_name="c", subcore_axis_name="s")
    parts = pl.kernel(
        _partials_body,
        out_type=jax.ShapeDtypeStruct((_NW, 2, _LANES), jnp.float32),
        mesh=mesh,
        scratch_types=[
            pltpu.VMEM((_N, _CHUNK), jnp.int32),
            pltpu.VMEM((_N, _CHUNK), jnp.float32),
            pltpu.VMEM((_CHUNK,), jnp.float32),
            pltpu.VMEM((_ROUND + 1, _TILE_R, _TILE_C), jnp.float32),
            pltpu.VMEM((_ROUND + 1, _TILE_R, _TILE_C), jnp.float32),
            pltpu.VMEM((2, _LANES), jnp.float32),
            pltpu.SemaphoreType.DMA,
            pltpu.SemaphoreType.DMA,
        ],
    )(lp, tgt, msk)

    out = pl.pallas_call(
        _finalize_body,
        out_shape=jax.ShapeDtypeStruct((1, 1), jnp.float32),
    )(parts)
    return out[0, 0]


# speedup vs baseline: 1.0572x; 1.0572x over previous
import jax
import jax.numpy as jnp
from jax import lax
from jax.experimental import pallas as pl
from jax.experimental.pallas import tpu as pltpu
from jax.experimental.pallas import tpu_sc as plsc

_N, _L, _V = 8, 512, 32000
_B = _N * _L
_NC = 2
_NS = 16
_NW = _NC * _NS
_CHUNK = _B // _NW
_LANES = 16
_TILE_R, _TILE_C = 8, 128
_TCOLS = _V // _TILE_C
_NTILES = (_B // _TILE_R) * _TCOLS
_ROUND = 32
_NROUND = _CHUNK // _ROUND


def _partials_body(lp_hbm, tgt_hbm, msk_hbm, out_hbm,
                   tgt_v, msk_v, idx_v, val_v, gran_a, gran_b, part_v,
                   sem_a, sem_b):
    wid = lax.axis_index("s") * _NC + lax.axis_index("c")
    n = wid & (_N - 1)
    lwin = lax.shift_right_logical(wid, 3)
    col_base = pl.multiple_of(lwin * _CHUNK, _CHUNK)
    base = n * _L + lwin * _CHUNK

    pltpu.sync_copy(tgt_hbm.at[:, pl.ds(col_base, _CHUNK)], tgt_v)
    pltpu.sync_copy(msk_hbm.at[:, pl.ds(col_base, _CHUNK)], msk_v)

    lane = lax.iota(jnp.int32, _LANES)
    for rnd in range(_NROUND):
        for j in range(_ROUND // _LANES):
            g = rnd * _ROUND + j * _LANES
            tv = tgt_v[n, pl.ds(g, _LANES)]
            row = base + g + lane
            idx_v[rnd, pl.ds(j * _LANES, _LANES)] = (
                lax.shift_right_logical(row, 3) * _TCOLS
                + lax.shift_right_logical(tv, 7))

    bufs = (gran_a, gran_b)
    sems = (sem_a, sem_b)
    def fire(rnd):
        return pltpu.async_copy(lp_hbm.at[idx_v.at[rnd]],
                                bufs[rnd % 2].at[pl.ds(0, _ROUND)],
                                sems[rnd % 2])

    copies = {0: fire(0), 1: fire(1)}
    for rnd in range(_NROUND):
        copies[rnd].wait()
        buf = bufs[rnd % 2]
        for j in range(_ROUND // _LANES):
            g = rnd * _ROUND + j * _LANES
            tv = tgt_v[n, pl.ds(g, _LANES)]
            cols = tv & (_TILE_C - 1)
            vacc = jnp.zeros((_LANES,), jnp.float32)
            for i in range(_LANES):
                p = j * _LANES + i
                start = cols[i] - i
                v16 = buf[p, (g + i) & (_TILE_R - 1), pl.ds(start, _LANES)]
                vacc = jnp.where(lane == i, v16, vacc)
            val_v[pl.ds(g, _LANES)] = vacc
        if rnd + 2 < _NROUND:
            copies[rnd + 2] = fire(rnd + 2)

    accw = jnp.zeros((_LANES,), jnp.float32)
    accm = jnp.zeros((_LANES,), jnp.float32)
    for j in range(_CHUNK // _LANES):
        sl = pl.ds(j * _LANES, _LANES)
        m = msk_v[n, sl]
        accw = accw + val_v[sl] * m
        accm = accm + m
    part_v[0, :] = accw
    part_v[1, :] = accm
    pltpu.sync_copy(part_v, out_hbm.at[wid])


def _finalize_body(parts_ref, out_ref):
    ws = jnp.sum(parts_ref[:, 0, :])
    ms = jnp.sum(parts_ref[:, 1, :])
    out_ref[...] = jnp.full((1, 1), -ws / (ms + 1e-6), jnp.float32)


def kernel(logprobs, target, mask):
    lp = (logprobs.reshape(_B // _TILE_R, _TILE_R, _TCOLS, _TILE_C)
          .transpose(0, 2, 1, 3)
          .reshape(_NTILES, _TILE_R, _TILE_C))
    tgt = target.astype(jnp.int32)
    msk = mask.astype(jnp.float32)

    mesh = plsc.VectorSubcoreMesh(core_axis_name="c", subcore_axis_name="s")
    parts = pl.kernel(
        _partials_body,
        out_type=jax.ShapeDtypeStruct((_NW, 2, _LANES), jnp.float32),
        mesh=mesh,
        scratch_types=[
            pltpu.VMEM((_N, _CHUNK), jnp.int32),
            pltpu.VMEM((_N, _CHUNK), jnp.float32),
            pltpu.VMEM((_NROUND, _ROUND), jnp.int32),
            pltpu.VMEM((_CHUNK,), jnp.float32),
            pltpu.VMEM((_ROUND + 1, _TILE_R, _TILE_C), jnp.float32),
            pltpu.VMEM((_ROUND + 1, _TILE_R, _TILE_C), jnp.float32),
            pltpu.VMEM((2, _LANES), jnp.float32),
            pltpu.SemaphoreType.DMA,
            pltpu.SemaphoreType.DMA,
        ],
    )(lp, tgt, msk)

    out = pl.pallas_call(
        _finalize_body,
        out_shape=jax.ShapeDtypeStruct((1, 1), jnp.float32),
    )(parts)
    return out[0, 0]


# speedup vs baseline: 1.3160x; 1.2448x over previous
import jax
import jax.numpy as jnp
from jax import lax
from jax.experimental import pallas as pl
from jax.experimental.pallas import tpu as pltpu
from jax.experimental.pallas import tpu_sc as plsc

_N, _L, _V = 8, 512, 32000
_B = _N * _L
_NC = 2
_NS = 16
_NW = _NC * _NS
_CHUNK = _B // _NW
_LANES = 16
_TILE_R, _TILE_C = 8, 128
_TCOLS = _V // _TILE_C
_NROWS = (_B // _TILE_R) * _TCOLS * _TILE_R


def _partials_body(lp_hbm, tgt_hbm, msk_hbm, out_hbm,
                   tgt_v, msk_v, idx_v, val_v, gran_v, part_v, sem_a):
    wid = lax.axis_index("s") * _NC + lax.axis_index("c")
    n = wid & (_N - 1)
    lwin = lax.shift_right_logical(wid, 3)
    col_base = pl.multiple_of(lwin * _CHUNK, _CHUNK)
    base = n * _L + lwin * _CHUNK

    pltpu.sync_copy(tgt_hbm.at[:, pl.ds(col_base, _CHUNK)], tgt_v)
    pltpu.sync_copy(msk_hbm.at[:, pl.ds(col_base, _CHUNK)], msk_v)

    lane = lax.iota(jnp.int32, _LANES)
    for j in range(_CHUNK // _LANES):
        g = j * _LANES
        tv = tgt_v[n, pl.ds(g, _LANES)]
        row = base + g + lane
        idx_v[pl.ds(g, _LANES)] = (
            (lax.shift_right_logical(row, 3) * _TCOLS
             + lax.shift_right_logical(tv, 7)) * _TILE_R + (row & (_TILE_R - 1)))

    pltpu.async_copy(lp_hbm.at[idx_v], gran_v.at[pl.ds(0, _CHUNK)],
                     sem_a).wait()
    for j in range(_CHUNK // _LANES):
        g = j * _LANES
        tv = tgt_v[n, pl.ds(g, _LANES)]
        cols = tv & (_TILE_C - 1)
        vacc = jnp.zeros((_LANES,), jnp.float32)
        for i in range(_LANES):
            start = cols[i] - i
            v16 = gran_v[g + i, pl.ds(start, _LANES)]
            vacc = jnp.where(lane == i, v16, vacc)
        val_v[pl.ds(g, _LANES)] = vacc

    accw = jnp.zeros((_LANES,), jnp.float32)
    accm = jnp.zeros((_LANES,), jnp.float32)
    for j in range(_CHUNK // _LANES):
        sl = pl.ds(j * _LANES, _LANES)
        m = msk_v[n, sl]
        accw = accw + val_v[sl] * m
        accm = accm + m
    part_v[0, :] = accw
    part_v[1, :] = accm
    pltpu.sync_copy(part_v, out_hbm.at[wid])


def _finalize_body(parts_ref, out_ref):
    ws = jnp.sum(parts_ref[:, 0, :])
    ms = jnp.sum(parts_ref[:, 1, :])
    out_ref[...] = jnp.full((1, 1), -ws / (ms + 1e-6), jnp.float32)


def kernel(logprobs, target, mask):
    lp = (logprobs.reshape(_B // _TILE_R, _TILE_R, _TCOLS, _TILE_C)
          .transpose(0, 2, 1, 3)
          .reshape(_NROWS, _TILE_C))
    tgt = target.astype(jnp.int32)
    msk = mask.astype(jnp.float32)

    mesh = plsc.VectorSubcoreMesh(core_axis_name="c", subcore_axis_name="s")
    parts = pl.kernel(
        _partials_body,
        out_type=jax.ShapeDtypeStruct((_NW, 2, _LANES), jnp.float32),
        mesh=mesh,
        scratch_types=[
            pltpu.VMEM((_N, _CHUNK), jnp.int32),
            pltpu.VMEM((_N, _CHUNK), jnp.float32),
            pltpu.VMEM((_CHUNK,), jnp.int32),
            pltpu.VMEM((_CHUNK,), jnp.float32),
            pltpu.VMEM((_CHUNK + 1, _TILE_C), jnp.float32),
            pltpu.VMEM((2, _LANES), jnp.float32),
            pltpu.SemaphoreType.DMA,
        ],
    )(lp, tgt, msk)

    out = pl.pallas_call(
        _finalize_body,
        out_shape=jax.ShapeDtypeStruct((1, 1), jnp.float32),
    )(parts)
    return out[0, 0]


# speedup vs baseline: 1.3368x; 1.0158x over previous
import jax
import jax.numpy as jnp
from jax import lax
from jax.experimental import pallas as pl
from jax.experimental.pallas import tpu as pltpu
from jax.experimental.pallas import tpu_sc as plsc

_N, _L, _V = 8, 512, 32000
_B = _N * _L
_NC = 2
_NS = 16
_NW = _NC * _NS
_CHUNK = _B // _NW
_LANES = 16
_TILE_R, _TILE_C = 8, 128
_TCOLS = _V // _TILE_C
_NROWS = (_B // _TILE_R) * _TCOLS * _TILE_R


def _partials_body(lp_hbm, tgt_hbm, msk_hbm, out_hbm,
                   tgt_v, msk_v, idx_v, gran_v, part_v, sem_a):
    wid = lax.axis_index("s") * _NC + lax.axis_index("c")
    n = wid & (_N - 1)
    lwin = lax.shift_right_logical(wid, 3)
    col_base = pl.multiple_of(lwin * _CHUNK, _CHUNK)
    base = n * _L + lwin * _CHUNK

    pltpu.sync_copy(tgt_hbm.at[:, pl.ds(col_base, _CHUNK)], tgt_v)
    pltpu.sync_copy(msk_hbm.at[:, pl.ds(col_base, _CHUNK)], msk_v)

    lane = lax.iota(jnp.int32, _LANES)
    for j in range(_CHUNK // _LANES):
        g = j * _LANES
        tv = tgt_v[n, pl.ds(g, _LANES)]
        row = base + g + lane
        idx_v[pl.ds(g, _LANES)] = (
            (lax.shift_right_logical(row, 3) * _TCOLS
             + lax.shift_right_logical(tv, 7)) * _TILE_R + (row & (_TILE_R - 1)))

    pltpu.async_copy(lp_hbm.at[idx_v], gran_v.at[pl.ds(0, _CHUNK)],
                     sem_a).wait()

    accw = jnp.zeros((_LANES,), jnp.float32)
    accm = jnp.zeros((_LANES,), jnp.float32)
    for j in range(_CHUNK // _LANES):
        g = j * _LANES
        tv = tgt_v[n, pl.ds(g, _LANES)]
        cols = tv & (_TILE_C - 1)
        vacc = jnp.zeros((_LANES,), jnp.float32)
        for i in range(_LANES):
            start = cols[i] - i
            v16 = gran_v[g + i, pl.ds(start, _LANES)]
            vacc = jnp.where(lane == i, v16, vacc)
        m = msk_v[n, pl.ds(g, _LANES)]
        accw = accw + vacc * m
        accm = accm + m
    part_v[0, :] = accw
    part_v[1, :] = accm
    pltpu.sync_copy(part_v, out_hbm.at[wid])


def _finalize_body(parts_ref, out_ref):
    ws = jnp.sum(parts_ref[:, 0, :])
    ms = jnp.sum(parts_ref[:, 1, :])
    out_ref[...] = jnp.full((1, 1), -ws / (ms + 1e-6), jnp.float32)


def kernel(logprobs, target, mask):
    lp = (logprobs.reshape(_B // _TILE_R, _TILE_R, _TCOLS, _TILE_C)
          .transpose(0, 2, 1, 3)
          .reshape(_NROWS, _TILE_C))
    tgt = target.astype(jnp.int32)
    msk = mask.astype(jnp.float32)

    mesh = plsc.VectorSubcoreMesh(core_axis_name="c", subcore_axis_name="s")
    parts = pl.kernel(
        _partials_body,
        out_type=jax.ShapeDtypeStruct((_NW, 2, _LANES), jnp.float32),
        mesh=mesh,
        scratch_types=[
            pltpu.VMEM((_N, _CHUNK), jnp.int32),
            pltpu.VMEM((_N, _CHUNK), jnp.float32),
            pltpu.VMEM((_CHUNK,), jnp.int32),
            pltpu.VMEM((_CHUNK + 1, _TILE_C), jnp.float32),
            pltpu.VMEM((2, _LANES), jnp.float32),
            pltpu.SemaphoreType.DMA,
        ],
    )(lp, tgt, msk)

    out = pl.pallas_call(
        _finalize_body,
        out_shape=jax.ShapeDtypeStruct((1, 1), jnp.float32),
    )(parts)
    return out[0, 0]
